# baseline (speedup 1.0000x reference)
import functools

import jax
import jax.numpy as jnp
from jax import lax
from jax.experimental import pallas as pl
from jax.experimental.pallas import tpu as pltpu
from jax.experimental.pallas import tpu_sc as plsc

N = 10000
E = 320000
D = 128

NUM_CORES = 2
NUM_SUBCORES = 16
NW = NUM_CORES * NUM_SUBCORES

CHUNK = 128
IB = 8
NBLK = 10
CHUNKS = IB * NBLK
EDGES_PER_W = CHUNKS * CHUNK
E_PAD = NW * EDGES_PER_W

ACC_ROWS = 10240
ROWS_PER_SUB = ACC_ROWS // NUM_SUBCORES
ZROWS = 32


def _agg_body(x_hbm, src_hbm, dst_hbm, out_hbm, sib0, sib1, dib0, dib1,
              rb0, rb1, zbuf, acc, is0, is1, gs0, gs1, as0, as1):
    cid = lax.axis_index("c")
    sid = lax.axis_index("s")
    wid = cid * NUM_SUBCORES + sid

    def idx_fetch(b, sib, dib, isem):
        pltpu.make_async_copy(src_hbm.at[wid, b], sib, isem).start()
        pltpu.make_async_copy(dst_hbm.at[wid, b], dib, isem).start()

    def idx_wait(sib, dib, isem):
        pltpu.make_async_copy(src_hbm.at[wid, 0], sib, isem).wait()
        pltpu.make_async_copy(dst_hbm.at[wid, 0], dib, isem).wait()

    idx_fetch(0, sib0, dib0, is0)
    idx_fetch(1, sib1, dib1, is1)

    @pl.loop(0, ZROWS)
    def _(r):
        @pl.loop(0, D, step=16)
        def _(c):
            zbuf[r, pl.ds(c, 16)] = jnp.zeros((16,), jnp.float32)

    zbase = sid * ROWS_PER_SUB

    @pl.loop(0, ROWS_PER_SUB // ZROWS)
    def _(i):
        pltpu.sync_copy(zbuf, acc.at[pl.ds(zbase + i * ZROWS, ZROWS)])

    plsc.subcore_barrier()

    rbs, gss, ass = (rb0, rb1), (gs0, gs1), (as0, as1)

    def gather(sib, k, p):
        pltpu.make_async_copy(x_hbm.at[sib.at[k]], rbs[p], gss[p]).start()

    def do_block(b, sib, dib, isem):
        idx_wait(sib, dib, isem)
        gather(sib, 0, 0)
        for k in range(IB):
            p = k % 2
            if k + 1 < IB:
                gather(sib, k + 1, (k + 1) % 2)
            pltpu.make_async_copy(x_hbm.at[sib.at[0]], rbs[p], gss[p]).wait()
            pltpu.async_copy(rbs[p], acc.at[dib.at[k]], ass[p], add=True)
            pltpu.make_async_copy(rbs[p], acc.at[dib.at[0]], ass[p]).wait()
        @pl.when(b + 2 < NBLK)
        def _():
            idx_fetch(b + 2, sib, dib, isem)

    @pl.loop(0, NBLK, step=2)
    def _(b):
        do_block(b, sib0, dib0, is0)
        do_block(b + 1, sib1, dib1, is1)

    plsc.subcore_barrier()

    pltpu.sync_copy(acc.at[pl.ds(zbase, ROWS_PER_SUB)],
                    out_hbm.at[cid, pl.ds(zbase, ROWS_PER_SUB)])


def _sc_agg(x, src3, dst3):
    mesh = plsc.VectorSubcoreMesh(core_axis_name="c", subcore_axis_name="s")
    k = pl.kernel(
        _agg_body,
        mesh=mesh,
        out_type=jax.ShapeDtypeStruct((NUM_CORES, ACC_ROWS, D), jnp.float32),
        scratch_types=[
            pltpu.VMEM((IB, CHUNK), jnp.int32),
            pltpu.VMEM((IB, CHUNK), jnp.int32),
            pltpu.VMEM((IB, CHUNK), jnp.int32),
            pltpu.VMEM((IB, CHUNK), jnp.int32),
            pltpu.VMEM((CHUNK, D), jnp.float32),
            pltpu.VMEM((CHUNK, D), jnp.float32),
            pltpu.VMEM((ZROWS, D), jnp.float32),
            pltpu.VMEM_SHARED((ACC_ROWS, D), jnp.float32),
            pltpu.SemaphoreType.DMA,
            pltpu.SemaphoreType.DMA,
            pltpu.SemaphoreType.DMA,
            pltpu.SemaphoreType.DMA,
            pltpu.SemaphoreType.DMA,
            pltpu.SemaphoreType.DMA,
        ],
    )
    return k(x, src3, dst3)


def _mlp_body(x_ref, p0_ref, p1_ref, w_ref, b_ref, o_ref):
    s = x_ref[...] + p0_ref[...] + p1_ref[...]
    acc = lax.dot_general(s, w_ref[...], (((1,), (0,)), ((), ())),
                          preferred_element_type=jnp.float32,
                          precision=lax.Precision.HIGHEST)
    o_ref[...] = jnp.maximum(acc + b_ref[...], 0.0)


_BLK = 1000


def _mlp(x, parts, w, b):
    grid = (N // _BLK,)
    return pl.pallas_call(
        _mlp_body,
        grid=grid,
        in_specs=[
            pl.BlockSpec((_BLK, D), lambda i: (i, 0)),
            pl.BlockSpec((_BLK, D), lambda i: (i, 0)),
            pl.BlockSpec((_BLK, D), lambda i: (i, 0)),
            pl.BlockSpec((D, D), lambda i: (0, 0)),
            pl.BlockSpec((1, D), lambda i: (0, 0)),
        ],
        out_specs=pl.BlockSpec((_BLK, D), lambda i: (i, 0)),
        out_shape=jax.ShapeDtypeStruct((N, D), jnp.float32),
    )(x, parts[0], parts[1], w, b)


def kernel(node_emb, edge_index, W1, b1, W2, b2):
    src = edge_index[0].astype(jnp.int32)
    dst = edge_index[1].astype(jnp.int32)
    pad = E_PAD - E
    src3 = jnp.concatenate([src, jnp.zeros((pad,), jnp.int32)]).reshape(
        NW, NBLK, IB, CHUNK)
    dst3 = jnp.concatenate([dst, jnp.full((pad,), N, jnp.int32)]).reshape(
        NW, NBLK, IB, CHUNK)
    b1r = b1.reshape(1, D)
    b2r = b2.reshape(1, D)

    p1 = _sc_agg(node_emb, src3, dst3)
    h1 = _mlp(node_emb, p1, W1, b1r)
    p2 = _sc_agg(h1, src3, dst3)
    return _mlp(h1, p2, W2, b2r)


# speedup vs baseline: 1.3541x; 1.3541x over previous
import functools

import jax
import jax.numpy as jnp
from jax import lax
from jax.experimental import pallas as pl
from jax.experimental.pallas import tpu as pltpu
from jax.experimental.pallas import tpu_sc as plsc

N = 10000
E = 320000
D = 128

NUM_CORES = 2
NUM_SUBCORES = 16
NW = NUM_CORES * NUM_SUBCORES

CHUNK = 128
CHUNKS = -(-E // (NW * CHUNK))
EDGES_PER_W = CHUNKS * CHUNK
E_PAD = NW * EDGES_PER_W

ACC_ROWS = 10240
ROWS_PER_SUB = ACC_ROWS // NUM_SUBCORES
ZROWS = 64


def _agg_body(x_hbm, src_hbm, dst_hbm, out_hbm, src_v, dst_v, rows_v, zbuf, acc):
    cid = lax.axis_index("c")
    sid = lax.axis_index("s")
    wid = cid * NUM_SUBCORES + sid

    @pl.loop(0, ZROWS)
    def _(r):
        @pl.loop(0, zbuf.shape[1], step=16)
        def _(c):
            zbuf[r, pl.ds(c, 16)] = jnp.zeros((16,), jnp.float32)

    zbase = sid * ROWS_PER_SUB

    @pl.loop(0, ROWS_PER_SUB // ZROWS)
    def _(i):
        pltpu.sync_copy(zbuf, acc.at[pl.ds(zbase + i * ZROWS, ZROWS)])

    pltpu.sync_copy(src_hbm.at[wid], src_v)
    pltpu.sync_copy(dst_hbm.at[wid], dst_v)

    plsc.subcore_barrier()

    @pl.loop(0, CHUNKS)
    def _(j):
        pltpu.sync_copy(x_hbm.at[src_v.at[j]], rows_v)
        pltpu.sync_copy(rows_v, acc.at[dst_v.at[j]], add=True)

    plsc.subcore_barrier()

    pltpu.sync_copy(acc.at[pl.ds(zbase, ROWS_PER_SUB)],
                    out_hbm.at[cid, pl.ds(zbase, ROWS_PER_SUB)])


def _sc_agg(x, src3, dst3):
    mesh = plsc.VectorSubcoreMesh(core_axis_name="c", subcore_axis_name="s")
    k = pl.kernel(
        _agg_body,
        mesh=mesh,
        out_type=jax.ShapeDtypeStruct((NUM_CORES, ACC_ROWS, D), jnp.float32),
        scratch_types=[
            pltpu.VMEM((CHUNKS, CHUNK), jnp.int32),
            pltpu.VMEM((CHUNKS, CHUNK), jnp.int32),
            pltpu.VMEM((CHUNK, D), jnp.float32),
            pltpu.VMEM((ZROWS, D), jnp.float32),
            pltpu.VMEM_SHARED((ACC_ROWS, D), jnp.float32),
        ],
    )
    return k(x, src3, dst3)


def _mlp_body(x_ref, p0_ref, p1_ref, w_ref, b_ref, o_ref):
    s = x_ref[...] + p0_ref[...] + p1_ref[...]
    acc = lax.dot_general(s, w_ref[...], (((1,), (0,)), ((), ())),
                          preferred_element_type=jnp.float32,
                          precision=lax.Precision.HIGHEST)
    o_ref[...] = jnp.maximum(acc + b_ref[...], 0.0)


_BLK = 1000


def _mlp(x, parts, w, b):
    grid = (N // _BLK,)
    return pl.pallas_call(
        _mlp_body,
        grid=grid,
        in_specs=[
            pl.BlockSpec((_BLK, D), lambda i: (i, 0)),
            pl.BlockSpec((_BLK, D), lambda i: (i, 0)),
            pl.BlockSpec((_BLK, D), lambda i: (i, 0)),
            pl.BlockSpec((D, D), lambda i: (0, 0)),
            pl.BlockSpec((1, D), lambda i: (0, 0)),
        ],
        out_specs=pl.BlockSpec((_BLK, D), lambda i: (i, 0)),
        out_shape=jax.ShapeDtypeStruct((N, D), jnp.float32),
    )(x, parts[0], parts[1], w, b)


def kernel(node_emb, edge_index, W1, b1, W2, b2):
    src = edge_index[0].astype(jnp.int32)
    dst = edge_index[1].astype(jnp.int32)
    pad = E_PAD - E
    src3 = jnp.concatenate([src, jnp.zeros((pad,), jnp.int32)]).reshape(
        NW, CHUNKS, CHUNK)
    dst3 = jnp.concatenate([dst, jnp.full((pad,), N, jnp.int32)]).reshape(
        NW, CHUNKS, CHUNK)
    b1r = b1.reshape(1, D)
    b2r = b2.reshape(1, D)

    p1 = _sc_agg(node_emb, src3, dst3)
    h1 = _mlp(node_emb, p1, W1, b1r)
    p2 = _sc_agg(h1, src3, dst3)
    return _mlp(h1, p2, W2, b2r)


# speedup vs baseline: 1.3556x; 1.0011x over previous
import jax
import jax.numpy as jnp
from jax import lax
from jax.experimental import pallas as pl
from jax.experimental.pallas import tpu as pltpu
from jax.experimental.pallas import tpu_sc as plsc

N = 10000
E = 320000
D = 128

NUM_CORES = 2
NUM_SUBCORES = 16
NW = NUM_CORES * NUM_SUBCORES

CHUNK = 128
CHUNKS = -(-E // (NW * CHUNK))
EDGES_PER_W = CHUNKS * CHUNK
E_PAD = NW * EDGES_PER_W

ACC_ROWS = 10240
ROWS_PER_SUB = ACC_ROWS // NUM_SUBCORES
ZROWS = 64


def _agg_body(x_hbm, src_hbm, dst_hbm, out_hbm, src_v, dst_v, rows_v, zbuf, acc):
    cid = lax.axis_index("c")
    sid = lax.axis_index("s")
    wid = cid * NUM_SUBCORES + sid

    @pl.loop(0, ZROWS)
    def _(r):
        @pl.loop(0, zbuf.shape[1], step=16)
        def _(c):
            zbuf[r, pl.ds(c, 16)] = jnp.zeros((16,), jnp.float32)

    zbase = sid * ROWS_PER_SUB

    @pl.loop(0, ROWS_PER_SUB // ZROWS)
    def _(i):
        pltpu.sync_copy(zbuf, acc.at[pl.ds(zbase + i * ZROWS, ZROWS)])

    pltpu.sync_copy(src_hbm.at[wid], src_v)
    pltpu.sync_copy(dst_hbm.at[wid], dst_v)

    plsc.subcore_barrier()

    @pl.loop(0, CHUNKS)
    def _(j):
        pltpu.sync_copy(x_hbm.at[src_v.at[j]], rows_v)
        pltpu.sync_copy(rows_v, acc.at[dst_v.at[j]], add=True)

    plsc.subcore_barrier()

    pltpu.sync_copy(acc.at[pl.ds(zbase, ROWS_PER_SUB)],
                    out_hbm.at[cid, pl.ds(zbase, ROWS_PER_SUB)])


def _sc_agg(x, src3, dst3):
    mesh = plsc.VectorSubcoreMesh(core_axis_name="c", subcore_axis_name="s")
    k = pl.kernel(
        _agg_body,
        mesh=mesh,
        out_type=jax.ShapeDtypeStruct((NUM_CORES, ACC_ROWS, D), jnp.float32),
        scratch_types=[
            pltpu.VMEM((CHUNKS, CHUNK), jnp.int32),
            pltpu.VMEM((CHUNKS, CHUNK), jnp.int32),
            pltpu.VMEM((CHUNK, D), jnp.float32),
            pltpu.VMEM((ZROWS, D), jnp.float32),
            pltpu.VMEM_SHARED((ACC_ROWS, D), jnp.float32),
        ],
    )
    return k(x, src3, dst3)


def _mlp_body(x_ref, p0_ref, p1_ref, w_ref, b_ref, o_ref):
    s = x_ref[...] + p0_ref[...] + p1_ref[...]
    acc = lax.dot_general(s, w_ref[...], (((1,), (0,)), ((), ())),
                          preferred_element_type=jnp.float32,
                          precision=lax.Precision.HIGHEST)
    o_ref[...] = jnp.maximum(acc + b_ref[...], 0.0)


_BLK = 1000


def _mlp(x, parts, w, b):
    grid = (N // _BLK,)
    return pl.pallas_call(
        _mlp_body,
        grid=grid,
        in_specs=[
            pl.BlockSpec((_BLK, D), lambda i: (i, 0)),
            pl.BlockSpec((_BLK, D), lambda i: (i, 0)),
            pl.BlockSpec((_BLK, D), lambda i: (i, 0)),
            pl.BlockSpec((D, D), lambda i: (0, 0)),
            pl.BlockSpec((1, D), lambda i: (0, 0)),
        ],
        out_specs=pl.BlockSpec((_BLK, D), lambda i: (i, 0)),
        out_shape=jax.ShapeDtypeStruct((N, D), jnp.float32),
    )(x, parts[0], parts[1], w, b)


def kernel(node_emb, edge_index, W1, b1, W2, b2):
    src = edge_index[0].astype(jnp.int32)
    dst = edge_index[1].astype(jnp.int32)
    pad = E_PAD - E
    src3 = jnp.concatenate([src, jnp.zeros((pad,), jnp.int32)]).reshape(
        NW, CHUNKS, CHUNK)
    dst3 = jnp.concatenate([dst, jnp.full((pad,), N, jnp.int32)]).reshape(
        NW, CHUNKS, CHUNK)
    b1r = b1.reshape(1, D)
    b2r = b2.reshape(1, D)

    p1 = _sc_agg(node_emb, src3, dst3)
    h1 = _mlp(node_emb, p1, W1, b1r)
    p2 = _sc_agg(h1, src3, dst3)
    return _mlp(h1, p2, W2, b2r)


# speedup vs baseline: 1.3651x; 1.0070x over previous
import jax
import jax.numpy as jnp
from jax import lax
from jax.experimental import pallas as pl
from jax.experimental.pallas import tpu as pltpu
from jax.experimental.pallas import tpu_sc as plsc

N = 10000
E = 320000
D = 128

NUM_CORES = 2
NUM_SUBCORES = 16
NW = NUM_CORES * NUM_SUBCORES

CHUNK = 128
CHUNKS = -(-E // (NW * CHUNK))
EDGES_PER_W = CHUNKS * CHUNK
E_PAD = NW * EDGES_PER_W

ACC_ROWS = 10240
ROWS_PER_SUB = ACC_ROWS // NUM_SUBCORES
ZROWS = 64


def _agg_body(x_hbm, src_hbm, dst_hbm, out_hbm, src_v, dst_v, rows_v, zbuf, acc,
              isem):
    cid = lax.axis_index("c")
    sid = lax.axis_index("s")
    wid = cid * NUM_SUBCORES + sid

    pltpu.make_async_copy(src_hbm.at[wid], src_v, isem).start()
    pltpu.make_async_copy(dst_hbm.at[wid], dst_v, isem).start()

    @pl.loop(0, ZROWS)
    def _(r):
        @pl.loop(0, zbuf.shape[1], step=16)
        def _(c):
            zbuf[r, pl.ds(c, 16)] = jnp.zeros((16,), jnp.float32)

    zbase = sid * ROWS_PER_SUB

    @pl.loop(0, ROWS_PER_SUB // ZROWS)
    def _(i):
        pltpu.sync_copy(zbuf, acc.at[pl.ds(zbase + i * ZROWS, ZROWS)])

    pltpu.make_async_copy(src_hbm.at[wid], src_v, isem).wait()
    pltpu.make_async_copy(dst_hbm.at[wid], dst_v, isem).wait()

    plsc.subcore_barrier()

    @pl.loop(0, CHUNKS)
    def _(j):
        pltpu.sync_copy(x_hbm.at[src_v.at[j]], rows_v)
        pltpu.sync_copy(rows_v, acc.at[dst_v.at[j]], add=True)

    plsc.subcore_barrier()

    pltpu.sync_copy(acc.at[pl.ds(zbase, ROWS_PER_SUB)],
                    out_hbm.at[cid, pl.ds(zbase, ROWS_PER_SUB)])


def _sc_agg(x, src3, dst3):
    mesh = plsc.VectorSubcoreMesh(core_axis_name="c", subcore_axis_name="s")
    k = pl.kernel(
        _agg_body,
        mesh=mesh,
        out_type=jax.ShapeDtypeStruct((NUM_CORES, ACC_ROWS, D), jnp.float32),
        scratch_types=[
            pltpu.VMEM((CHUNKS, CHUNK), jnp.int32),
            pltpu.VMEM((CHUNKS, CHUNK), jnp.int32),
            pltpu.VMEM((CHUNK, D), jnp.float32),
            pltpu.VMEM((ZROWS, D), jnp.float32),
            pltpu.VMEM_SHARED((ACC_ROWS, D), jnp.float32),
            pltpu.SemaphoreType.DMA,
        ],
    )
    return k(x, src3, dst3)


def _mlp_body(x_ref, p0_ref, p1_ref, w_ref, b_ref, o_ref):
    s = x_ref[...] + p0_ref[...] + p1_ref[...]
    acc = lax.dot_general(s, w_ref[...], (((1,), (0,)), ((), ())),
                          preferred_element_type=jnp.float32,
                          precision=lax.Precision.HIGHEST)
    o_ref[...] = jnp.maximum(acc + b_ref[...], 0.0)


_BLK = 1000


def _mlp(x, parts, w, b):
    grid = (N // _BLK,)
    return pl.pallas_call(
        _mlp_body,
        grid=grid,
        in_specs=[
            pl.BlockSpec((_BLK, D), lambda i: (i, 0)),
            pl.BlockSpec((_BLK, D), lambda i: (i, 0)),
            pl.BlockSpec((_BLK, D), lambda i: (i, 0)),
            pl.BlockSpec((D, D), lambda i: (0, 0)),
            pl.BlockSpec((1, D), lambda i: (0, 0)),
        ],
        out_specs=pl.BlockSpec((_BLK, D), lambda i: (i, 0)),
        out_shape=jax.ShapeDtypeStruct((N, D), jnp.float32),
    )(x, parts[0], parts[1], w, b)


def kernel(node_emb, edge_index, W1, b1, W2, b2):
    src = edge_index[0].astype(jnp.int32)
    dst = edge_index[1].astype(jnp.int32)
    pad = E_PAD - E
    src3 = jnp.concatenate([src, jnp.zeros((pad,), jnp.int32)]).reshape(
        NW, CHUNKS, CHUNK)
    dst3 = jnp.concatenate([dst, jnp.full((pad,), N, jnp.int32)]).reshape(
        NW, CHUNKS, CHUNK)
    b1r = b1.reshape(1, D)
    b2r = b2.reshape(1, D)

    p1 = _sc_agg(node_emb, src3, dst3)
    h1 = _mlp(node_emb, p1, W1, b1r)
    p2 = _sc_agg(h1, src3, dst3)
    return _mlp(h1, p2, W2, b2r)


# speedup vs baseline: 1.3715x; 1.0047x over previous
import jax
import jax.numpy as jnp
from jax import lax
from jax.experimental import pallas as pl
from jax.experimental.pallas import tpu as pltpu
from jax.experimental.pallas import tpu_sc as plsc

N = 10000
E = 320000
D = 128

NUM_CORES = 2
NUM_SUBCORES = 16
NW = NUM_CORES * NUM_SUBCORES

CHUNK = 128
CHUNKS = -(-E // (NW * CHUNK))
EDGES_PER_W = CHUNKS * CHUNK
E_PAD = NW * EDGES_PER_W

ACC_ROWS = 10240
ROWS_PER_SUB = ACC_ROWS // NUM_SUBCORES
ZROWS = 64


def _agg_body(x_hbm, src_hbm, dst_hbm, out_hbm, src_v, dst_v, rows_v, zbuf, acc,
              isem, zsem, gsem):
    cid = lax.axis_index("c")
    sid = lax.axis_index("s")
    wid = cid * NUM_SUBCORES + sid

    pltpu.make_async_copy(src_hbm.at[wid], src_v, isem).start()
    pltpu.make_async_copy(dst_hbm.at[wid], dst_v, isem).start()

    @pl.loop(0, ZROWS)
    def _(r):
        @pl.loop(0, zbuf.shape[1], step=16)
        def _(c):
            zbuf[r, pl.ds(c, 16)] = jnp.zeros((16,), jnp.float32)

    zbase = sid * ROWS_PER_SUB

    @pl.loop(0, ROWS_PER_SUB // ZROWS)
    def _(i):
        pltpu.make_async_copy(
            zbuf, acc.at[pl.ds(zbase + i * ZROWS, ZROWS)], zsem).start()

    pltpu.make_async_copy(src_hbm.at[wid], src_v, isem).wait()
    pltpu.make_async_copy(dst_hbm.at[wid], dst_v, isem).wait()

    pltpu.make_async_copy(x_hbm.at[src_v.at[0]], rows_v, gsem).start()

    @pl.loop(0, ROWS_PER_SUB // ZROWS)
    def _(i):
        pltpu.make_async_copy(
            zbuf, acc.at[pl.ds(zbase + i * ZROWS, ZROWS)], zsem).wait()

    plsc.subcore_barrier()

    pltpu.make_async_copy(x_hbm.at[src_v.at[0]], rows_v, gsem).wait()
    pltpu.sync_copy(rows_v, acc.at[dst_v.at[0]], add=True)

    @pl.loop(1, CHUNKS)
    def _(j):
        pltpu.sync_copy(x_hbm.at[src_v.at[j]], rows_v)
        pltpu.sync_copy(rows_v, acc.at[dst_v.at[j]], add=True)

    plsc.subcore_barrier()

    pltpu.sync_copy(acc.at[pl.ds(zbase, ROWS_PER_SUB)],
                    out_hbm.at[cid, pl.ds(zbase, ROWS_PER_SUB)])


def _sc_agg(x, src3, dst3):
    mesh = plsc.VectorSubcoreMesh(core_axis_name="c", subcore_axis_name="s")
    k = pl.kernel(
        _agg_body,
        mesh=mesh,
        out_type=jax.ShapeDtypeStruct((NUM_CORES, ACC_ROWS, D), jnp.float32),
        scratch_types=[
            pltpu.VMEM((CHUNKS, CHUNK), jnp.int32),
            pltpu.VMEM((CHUNKS, CHUNK), jnp.int32),
            pltpu.VMEM((CHUNK, D), jnp.float32),
            pltpu.VMEM((ZROWS, D), jnp.float32),
            pltpu.VMEM_SHARED((ACC_ROWS, D), jnp.float32),
            pltpu.SemaphoreType.DMA,
            pltpu.SemaphoreType.DMA,
            pltpu.SemaphoreType.DMA,
        ],
    )
    return k(x, src3, dst3)


def _mlp_body(x_ref, p0_ref, p1_ref, w_ref, b_ref, o_ref):
    s = x_ref[...] + p0_ref[...] + p1_ref[...]
    acc = lax.dot_general(s, w_ref[...], (((1,), (0,)), ((), ())),
                          preferred_element_type=jnp.float32,
                          precision=lax.Precision.HIGHEST)
    o_ref[...] = jnp.maximum(acc + b_ref[...], 0.0)


_BLK = 1000


def _mlp(x, parts, w, b):
    grid = (N // _BLK,)
    return pl.pallas_call(
        _mlp_body,
        grid=grid,
        in_specs=[
            pl.BlockSpec((_BLK, D), lambda i: (i, 0)),
            pl.BlockSpec((_BLK, D), lambda i: (i, 0)),
            pl.BlockSpec((_BLK, D), lambda i: (i, 0)),
            pl.BlockSpec((D, D), lambda i: (0, 0)),
            pl.BlockSpec((1, D), lambda i: (0, 0)),
        ],
        out_specs=pl.BlockSpec((_BLK, D), lambda i: (i, 0)),
        out_shape=jax.ShapeDtypeStruct((N, D), jnp.float32),
    )(x, parts[0], parts[1], w, b)


def kernel(node_emb, edge_index, W1, b1, W2, b2):
    src = edge_index[0].astype(jnp.int32)
    dst = edge_index[1].astype(jnp.int32)
    pad = E_PAD - E
    src3 = jnp.concatenate([src, jnp.zeros((pad,), jnp.int32)]).reshape(
        NW, CHUNKS, CHUNK)
    dst3 = jnp.concatenate([dst, jnp.full((pad,), N, jnp.int32)]).reshape(
        NW, CHUNKS, CHUNK)
    b1r = b1.reshape(1, D)
    b2r = b2.reshape(1, D)

    p1 = _sc_agg(node_emb, src3, dst3)
    h1 = _mlp(node_emb, p1, W1, b1r)
    p2 = _sc_agg(h1, src3, dst3)
    return _mlp(h1, p2, W2, b2r)


# speedup vs baseline: 1.3879x; 1.0119x over previous
import jax
import jax.numpy as jnp
from jax import lax
from jax.experimental import pallas as pl
from jax.experimental.pallas import tpu as pltpu
from jax.experimental.pallas import tpu_sc as plsc

N = 10000
E = 320000
D = 128

NUM_CORES = 2
NUM_SUBCORES = 16
NW = NUM_CORES * NUM_SUBCORES

CHUNK = 128
CHUNKS = -(-E // (NW * CHUNK))
EDGES_PER_W = CHUNKS * CHUNK
E_PAD = NW * EDGES_PER_W

ACC_ROWS = 10240
ROWS_PER_SUB = ACC_ROWS // NUM_SUBCORES
ZROWS = 64


def _agg_body(x_hbm, src_hbm, dst_hbm, out_hbm, src_v, dst_v, rows_v, zbuf, acc,
              isem, zsem, gsem):
    cid = lax.axis_index("c")
    sid = lax.axis_index("s")
    wid = cid * NUM_SUBCORES + sid

    pltpu.make_async_copy(src_hbm.at[wid], src_v, isem).start()
    pltpu.make_async_copy(dst_hbm.at[wid], dst_v, isem).start()

    @pl.loop(0, ZROWS)
    def _(r):
        @pl.loop(0, zbuf.shape[1], step=16)
        def _(c):
            zbuf[r, pl.ds(c, 16)] = jnp.zeros((16,), jnp.float32)

    zbase = sid * ROWS_PER_SUB

    @pl.loop(0, ROWS_PER_SUB // ZROWS)
    def _(i):
        pltpu.make_async_copy(
            zbuf, acc.at[pl.ds(zbase + i * ZROWS, ZROWS)], zsem).start()

    pltpu.make_async_copy(src_hbm.at[wid], src_v, isem).wait()
    pltpu.make_async_copy(dst_hbm.at[wid], dst_v, isem).wait()

    pltpu.make_async_copy(x_hbm.at[src_v.at[0]], rows_v, gsem).start()

    @pl.loop(0, ROWS_PER_SUB // ZROWS)
    def _(i):
        pltpu.make_async_copy(
            zbuf, acc.at[pl.ds(zbase + i * ZROWS, ZROWS)], zsem).wait()

    plsc.subcore_barrier()

    pltpu.make_async_copy(x_hbm.at[src_v.at[0]], rows_v, gsem).wait()
    pltpu.sync_copy(rows_v, acc.at[dst_v.at[0]], add=True)

    @pl.loop(1, CHUNKS)
    def _(j):
        pltpu.sync_copy(x_hbm.at[src_v.at[j]], rows_v)
        pltpu.sync_copy(rows_v, acc.at[dst_v.at[j]], add=True)

    plsc.subcore_barrier()

    pltpu.sync_copy(acc.at[pl.ds(zbase, ROWS_PER_SUB)],
                    out_hbm.at[cid, pl.ds(zbase, ROWS_PER_SUB)])


def _sc_agg(x, src3, dst3):
    mesh = plsc.VectorSubcoreMesh(core_axis_name="c", subcore_axis_name="s")
    k = pl.kernel(
        _agg_body,
        mesh=mesh,
        out_type=jax.ShapeDtypeStruct((NUM_CORES, ACC_ROWS, D), jnp.float32),
        scratch_types=[
            pltpu.VMEM((CHUNKS, CHUNK), jnp.int32),
            pltpu.VMEM((CHUNKS, CHUNK), jnp.int32),
            pltpu.VMEM((CHUNK, D), jnp.float32),
            pltpu.VMEM((ZROWS, D), jnp.float32),
            pltpu.VMEM_SHARED((ACC_ROWS, D), jnp.float32),
            pltpu.SemaphoreType.DMA,
            pltpu.SemaphoreType.DMA,
            pltpu.SemaphoreType.DMA,
        ],
    )
    return k(x, src3, dst3)


def _mlp_body(x_ref, p0_ref, p1_ref, w_ref, b_ref, o_ref):
    s = x_ref[...] + p0_ref[...] + p1_ref[...]
    acc = lax.dot_general(s, w_ref[...], (((1,), (0,)), ((), ())),
                          preferred_element_type=jnp.float32,
                          precision=lax.Precision.HIGHEST)
    o_ref[...] = jnp.maximum(acc + b_ref[...], 0.0)


_BLK = 2000


def _mlp(x, parts, w, b):
    grid = (N // _BLK,)
    return pl.pallas_call(
        _mlp_body,
        grid=grid,
        in_specs=[
            pl.BlockSpec((_BLK, D), lambda i: (i, 0)),
            pl.BlockSpec((_BLK, D), lambda i: (i, 0)),
            pl.BlockSpec((_BLK, D), lambda i: (i, 0)),
            pl.BlockSpec((D, D), lambda i: (0, 0)),
            pl.BlockSpec((1, D), lambda i: (0, 0)),
        ],
        out_specs=pl.BlockSpec((_BLK, D), lambda i: (i, 0)),
        out_shape=jax.ShapeDtypeStruct((N, D), jnp.float32),
    )(x, parts[0], parts[1], w, b)


def kernel(node_emb, edge_index, W1, b1, W2, b2):
    src = edge_index[0].astype(jnp.int32)
    dst = edge_index[1].astype(jnp.int32)
    pad = E_PAD - E
    src3 = jnp.concatenate([src, jnp.zeros((pad,), jnp.int32)]).reshape(
        NW, CHUNKS, CHUNK)
    dst3 = jnp.concatenate([dst, jnp.full((pad,), N, jnp.int32)]).reshape(
        NW, CHUNKS, CHUNK)
    b1r = b1.reshape(1, D)
    b2r = b2.reshape(1, D)

    p1 = _sc_agg(node_emb, src3, dst3)
    h1 = _mlp(node_emb, p1, W1, b1r)
    p2 = _sc_agg(h1, src3, dst3)
    return _mlp(h1, p2, W2, b2r)
